# baseline (speedup 1.0000x reference)
import functools

import jax
import jax.numpy as jnp
from jax import lax
from jax.experimental import pallas as pl
from jax.experimental.pallas import tpu as pltpu
from jax.experimental.pallas import tpu_sc as plsc


@functools.lru_cache(maxsize=None)
def _make_sc_gather(n_batch: int, seq: int, dim: int):
    info = plsc.get_sparse_core_info()
    nc, ns = info.num_cores, info.num_subcores
    nw = nc * ns
    assert n_batch % nw == 0
    b_per_w = n_batch // nw

    nbuf = 4
    assert b_per_w % nbuf == 0
    n_rounds = b_per_w // nbuf
    stride = 64
    main = (seq // 8) * 8

    mesh = plsc.VectorSubcoreMesh(core_axis_name="c", subcore_axis_name="s")

    @functools.partial(
        pl.kernel,
        mesh=mesh,
        out_type=jax.ShapeDtypeStruct((n_batch, seq, dim), jnp.float32),
        scratch_types=[
            pltpu.VMEM((b_per_w * stride,), jnp.int32),
            pltpu.VMEM((nbuf, main, dim), jnp.float32),
            pltpu.SemaphoreType.DMA,
            pltpu.SemaphoreType.DMA,
        ],
    )
    def sc_gather(table_hbm, idx_hbm, out_hbm, idx_v, rows_v, gsem, ssem):
        wid = lax.axis_index("s") * nc + lax.axis_index("c")
        base = wid * b_per_w
        pltpu.sync_copy(idx_hbm.at[pl.ds(base * stride, b_per_w * stride)], idx_v)

        def gather_start(c, b):
            pltpu.async_copy(
                table_hbm.at[idx_v.at[pl.ds(c * stride, main)]],
                rows_v.at[b], gsem)

        def gather_wait(b):
            pltpu.make_async_copy(
                out_hbm.at[0, pl.ds(0, main)], rows_v.at[b], gsem).wait()

        def scatter_start(c, b):
            pltpu.async_copy(
                rows_v.at[pl.ds(b, 1)],
                out_hbm.at[pl.ds(base + c, 1), pl.ds(0, main)], ssem)

        def scatter_wait(b):
            pltpu.make_async_copy(
                rows_v.at[pl.ds(b, 1)],
                out_hbm.at[pl.ds(0, 1), pl.ds(0, main)], ssem).wait()

        for b in range(nbuf):
            gather_start(b, b)

        def body(i, carry):
            for b in range(nbuf):
                cur = i * nbuf + b
                gather_wait(b)
                scatter_start(cur, b)
            for b in range(nbuf):
                nxt = (i + 1) * nbuf + b

                @pl.when(nxt < b_per_w)
                def _():
                    scatter_wait(b)
                    gather_start(nxt, b)

            return carry

        lax.fori_loop(0, n_rounds, body, 0)
        for b in range(nbuf):
            scatter_wait(b)

    return sc_gather


@functools.lru_cache(maxsize=None)
def _make_tc_patch(n_batch: int, seq: int, dim: int, vocab: int):
    main = (seq // 8) * 8
    tail = seq - main
    bblk = 128
    n_steps = n_batch // bblk
    rows_per_step = bblk * tail

    def patch_kernel(big_ref, table_ref, idx_ref, out_ref):
        del big_ref
        ids = idx_ref[0, 0, :]
        iota = lax.broadcasted_iota(jnp.int32, (rows_per_step, vocab), 1)
        onehot = (ids[:, None] == iota).astype(jnp.float32)
        rows = lax.dot_general(
            onehot, table_ref[...],
            (((1,), (0,)), ((), ())),
            precision=lax.Precision.HIGHEST,
            preferred_element_type=jnp.float32)
        rows8 = jnp.concatenate(
            [rows.reshape(bblk, tail, dim),
             jnp.zeros((bblk, 8 - tail, dim), jnp.float32)], axis=1)
        out_ref[...] = rows8

    return pl.pallas_call(
        patch_kernel,
        grid=(n_steps,),
        in_specs=[
            pl.BlockSpec((bblk, 8, dim), lambda i: (i, main // 8, 0)),
            pl.BlockSpec((vocab, dim), lambda i: (0, 0)),
            pl.BlockSpec((1, 1, rows_per_step), lambda i: (i, 0, 0)),
        ],
        out_specs=pl.BlockSpec((bblk, 8, dim), lambda i: (i, main // 8, 0)),
        out_shape=jax.ShapeDtypeStruct((n_batch, seq, dim), jnp.float32),
        input_output_aliases={0: 0},
    )


def kernel(input, embedding_weight):
    b, l = input.shape
    vocab, dim = embedding_weight.shape
    idx = input.astype(jnp.int32)
    idx_pad = jnp.pad(idx, ((0, 0), (0, 64 - l))).reshape(-1)
    out = _make_sc_gather(b, l, dim)(embedding_weight, idx_pad)
    main = (l // 8) * 8
    tail_idx = idx[:, main:].reshape(-1, 1, 128 * (l - main))
    out = _make_tc_patch(b, l, dim, vocab)(out, embedding_weight, tail_idx)
    return out


# speedup vs baseline: 1.1620x; 1.1620x over previous
import functools

import jax
import jax.numpy as jnp
from jax import lax
from jax.experimental import pallas as pl
from jax.experimental.pallas import tpu as pltpu
from jax.experimental.pallas import tpu_sc as plsc


@functools.lru_cache(maxsize=None)
def _make_sc_gather(n_batch: int, seq: int, dim: int):
    info = plsc.get_sparse_core_info()
    nc, ns = info.num_cores, info.num_subcores
    nw = nc * ns
    assert n_batch % nw == 0
    b_per_w = n_batch // nw

    nbuf = 4
    assert b_per_w % nbuf == 0
    n_rounds = b_per_w // nbuf
    stride = 64
    main = (seq // 8) * 8

    mesh = plsc.VectorSubcoreMesh(core_axis_name="c", subcore_axis_name="s")

    @functools.partial(
        pl.kernel,
        mesh=mesh,
        out_type=jax.ShapeDtypeStruct((n_batch, seq, dim), jnp.float32),
        scratch_types=[
            pltpu.VMEM((b_per_w * stride,), jnp.int32),
            pltpu.VMEM((nbuf, main, dim), jnp.float32),
            pltpu.SemaphoreType.DMA,
            pltpu.SemaphoreType.DMA,
        ],
    )
    def sc_gather(table_hbm, idx_hbm, out_hbm, idx_v, rows_v, gsem, ssem):
        wid = lax.axis_index("s") * nc + lax.axis_index("c")
        base = wid * b_per_w
        pltpu.sync_copy(idx_hbm.at[pl.ds(base * stride, b_per_w * stride)], idx_v)

        def gather_start(c, b):
            pltpu.async_copy(
                table_hbm.at[idx_v.at[pl.ds(c * stride, main)]],
                rows_v.at[b], gsem)

        def gather_wait(b):
            pltpu.make_async_copy(
                out_hbm.at[0, pl.ds(0, main)], rows_v.at[b], gsem).wait()

        def scatter_start(c, b):
            pltpu.async_copy(
                rows_v.at[pl.ds(b, 1)],
                out_hbm.at[pl.ds(base + c, 1), pl.ds(0, main)], ssem)

        def scatter_wait(b):
            pltpu.make_async_copy(
                rows_v.at[pl.ds(b, 1)],
                out_hbm.at[pl.ds(0, 1), pl.ds(0, main)], ssem).wait()

        for b in range(nbuf):
            gather_start(b, b)

        def body(i, carry):
            for b in range(nbuf):
                cur = i * nbuf + b
                gather_wait(b)
                scatter_start(cur, b)
            for b in range(nbuf):
                nxt = (i + 1) * nbuf + b

                @pl.when(nxt < b_per_w)
                def _():
                    scatter_wait(b)
                    gather_start(nxt, b)

            return carry

        lax.fori_loop(0, n_rounds, body, 0)
        for b in range(nbuf):
            scatter_wait(b)

    return sc_gather


@functools.lru_cache(maxsize=None)
def _make_tc_patch(n_batch: int, seq: int, dim: int, vocab: int):
    main = (seq // 8) * 8
    tail = seq - main
    bblk = 128
    n_steps = n_batch // bblk
    rows_per_step = bblk * tail

    def patch_kernel(table_ref, idx_ref, out_ref):
        ids = idx_ref[0, 0, :]
        iota = lax.broadcasted_iota(jnp.int32, (rows_per_step, vocab), 1)
        onehot = (ids[:, None] == iota).astype(jnp.float32)
        rows = lax.dot_general(
            onehot, table_ref[...],
            (((1,), (0,)), ((), ())),
            precision=lax.Precision.HIGHEST,
            preferred_element_type=jnp.float32)
        out_ref[...] = rows.reshape(bblk, tail, dim)

    return pl.pallas_call(
        patch_kernel,
        grid=(n_steps,),
        in_specs=[
            pl.BlockSpec((vocab, dim), lambda i: (0, 0)),
            pl.BlockSpec((1, 1, rows_per_step), lambda i: (i, 0, 0)),
        ],
        out_specs=pl.BlockSpec((bblk, tail, dim), lambda i: (i, 0, 0)),
        out_shape=jax.ShapeDtypeStruct((n_batch, tail, dim), jnp.float32),
    )


def kernel(input, embedding_weight):
    b, l = input.shape
    vocab, dim = embedding_weight.shape
    idx = input.astype(jnp.int32)
    idx_pad = jnp.pad(idx, ((0, 0), (0, 64 - l))).reshape(-1)
    out = _make_sc_gather(b, l, dim)(embedding_weight, idx_pad)
    main = (l // 8) * 8
    tail_idx = idx[:, main:].reshape(-1, 1, 128 * (l - main))
    tail_rows = _make_tc_patch(b, l, dim, vocab)(embedding_weight, tail_idx)
    return lax.dynamic_update_slice(out, tail_rows, (0, main, 0))


# speedup vs baseline: 2.1440x; 1.8452x over previous
import functools

import jax
import jax.numpy as jnp
from jax import lax
from jax.experimental import pallas as pl
from jax.experimental.pallas import tpu as pltpu
from jax.experimental.pallas import tpu_sc as plsc


@functools.lru_cache(maxsize=None)
def _make_gather(n_rows: int, dim: int):
    info = plsc.get_sparse_core_info()
    nc, ns = info.num_cores, info.num_subcores
    nw = nc * ns
    assert n_rows % (8 * nw) == 0
    b_per_w = n_rows // nw

    nbuf = 2
    chunk = 64
    assert b_per_w % (chunk * nbuf) == 0
    n_rounds = b_per_w // (chunk * nbuf)

    mesh = plsc.VectorSubcoreMesh(core_axis_name="c", subcore_axis_name="s")

    @functools.partial(
        pl.kernel,
        mesh=mesh,
        out_type=jax.ShapeDtypeStruct((n_rows, dim), jnp.float32),
        scratch_types=[
            pltpu.VMEM((b_per_w,), jnp.int32),
            pltpu.VMEM((nbuf, chunk, dim), jnp.float32),
            pltpu.SemaphoreType.DMA,
            pltpu.SemaphoreType.DMA,
        ],
    )
    def gather_kernel(table_hbm, idx_hbm, out_hbm, idx_v, rows_v, gsem, ssem):
        wid = lax.axis_index("s") * nc + lax.axis_index("c")
        base = wid * b_per_w
        pltpu.sync_copy(idx_hbm.at[pl.ds(base, b_per_w)], idx_v)

        def gather_start(c, b):
            pltpu.async_copy(
                table_hbm.at[idx_v.at[pl.ds(c * chunk, chunk)]],
                rows_v.at[b], gsem)

        def gather_wait(b):
            pltpu.make_async_copy(
                out_hbm.at[pl.ds(0, chunk)], rows_v.at[b], gsem).wait()

        def scatter_start(c, b):
            pltpu.async_copy(
                rows_v.at[b], out_hbm.at[pl.ds(base + c * chunk, chunk)], ssem)

        def scatter_wait(b):
            pltpu.make_async_copy(
                rows_v.at[b], out_hbm.at[pl.ds(0, chunk)], ssem).wait()

        for b in range(nbuf):
            gather_start(b, b)

        def body(i, carry):
            for b in range(nbuf):
                cur = i * nbuf + b
                gather_wait(b)
                scatter_start(cur, b)
            for b in range(nbuf):
                nxt = (i + 1) * nbuf + b

                @pl.when(nxt < n_rounds * nbuf)
                def _():
                    scatter_wait(b)
                    gather_start(nxt, b)

            return carry

        lax.fori_loop(0, n_rounds, body, 0)
        for b in range(nbuf):
            scatter_wait(b)

    return gather_kernel


def kernel(input, embedding_weight):
    b, l = input.shape
    _, dim = embedding_weight.shape
    idx = input.astype(jnp.int32).T.reshape(-1)
    out2d = _make_gather(idx.shape[0], dim)(embedding_weight, idx)
    return out2d.reshape(l, b, dim).transpose(1, 0, 2)
